# baseline (speedup 1.0000x reference)
import functools

import jax
import jax.numpy as jnp
from jax import lax
from jax.experimental import pallas as pl
from jax.experimental.pallas import tpu as pltpu
from jax.experimental.pallas import tpu_sc as plsc

N = 10000
E = 160000
D_IN = 128
H = 256
HH = 128
NC, NS = 2, 16
CH = 128
KCH = 80
HKCH = KCH // 2
E_PAD = NS * KCH * CH
NPAD = 10112
WR = 624
RB = 400
NB = N // RB

@functools.lru_cache(maxsize=None)
def _sc_mesh():
    return plsc.VectorSubcoreMesh(core_axis_name="c", subcore_axis_name="s",
                                  num_cores=NC, num_subcores=NS)


def _one_relation(p_hbm, srcoff_hbm, dst_hbm, zeros_hbm, out_hbm,
                  src_v, dst_v, rows, acc, semz, sems, ssems, hsems, c, s):
    rows0, rows1 = rows
    HC = CH // 2
    pltpu.sync_copy(zeros_hbm.at[pl.ds(0, CH)], rows0)
    zr = NPAD // NS
    zchunks = [(i * CH, CH) for i in range(zr // CH)] + [
        ((zr // CH) * CH, zr % CH)]
    for t, sz in zchunks:
        pltpu.async_copy(rows0.at[pl.ds(0, sz)],
                         acc.at[pl.ds(s * zr + t, sz)], semz)
    pltpu.sync_copy(dst_hbm.at[s], dst_v)
    for t, sz in zchunks:
        pltpu.make_async_copy(rows0.at[pl.ds(0, sz)],
                              acc.at[pl.ds(s * zr + t, sz)], semz).wait()
    plsc.subcore_barrier()

    def _gather(k, b):
        for q in range(2):
            pltpu.async_copy(p_hbm.at[src_v.at[k, pl.ds(q * HC, HC)]],
                             rows[b].at[pl.ds(q * HC, HC)],
                             (sems[b], hsems[b])[q])

    def _gwait(k, b):
        for q in range(2):
            pltpu.make_async_copy(p_hbm.at[src_v.at[k, pl.ds(q * HC, HC)]],
                                  rows[b].at[pl.ds(q * HC, HC)],
                                  (sems[b], hsems[b])[q]).wait()

    for h in range(2):
        pltpu.sync_copy(srcoff_hbm.at[c, s, pl.ds(h * HKCH, HKCH)], src_v)
        _gather(0, 0)

        def body(kk, carry):
            for b in range(2):
                k = 2 * kk + b
                _gwait(k, b)
                pltpu.async_copy(rows[b], acc.at[dst_v.at[h * HKCH + k]],
                                 ssems[b], add=True)

                @pl.when(k + 1 < HKCH)
                def _():
                    @pl.when(k >= 1)
                    def _():
                        pltpu.make_async_copy(
                            rows[1 - b],
                            acc.at[dst_v.at[h * HKCH + k - 1]],
                            ssems[1 - b]).wait()

                    _gather(k + 1, 1 - b)
            return carry

        lax.fori_loop(0, HKCH // 2, body, 0)
        for b, k in ((0, HKCH - 2), (1, HKCH - 1)):
            pltpu.make_async_copy(rows[b], acc.at[dst_v.at[h * HKCH + k]],
                                  ssems[b]).wait()
    plsc.subcore_barrier()
    chunks = [(i * CH, CH) for i in range(WR // CH)] + [
        ((WR // CH) * CH, WR % CH)]
    for idx, (t, sz) in enumerate(chunks):
        b = idx % 2
        if idx >= 2:
            pt, psz = chunks[idx - 2]
            pltpu.make_async_copy(
                rows[b].at[pl.ds(0, psz)],
                out_hbm.at[pl.ds(c * N + s * WR + pt, psz)], sems[b]).wait()
        pltpu.sync_copy(acc.at[pl.ds(s * WR + t, sz)],
                        rows[b].at[pl.ds(0, sz)])
        pltpu.async_copy(rows[b].at[pl.ds(0, sz)],
                         out_hbm.at[pl.ds(c * N + s * WR + t, sz)], sems[b])
    for idx in (len(chunks) - 2, len(chunks) - 1):
        t, sz = chunks[idx]
        pltpu.make_async_copy(
            rows[idx % 2].at[pl.ds(0, sz)],
            out_hbm.at[pl.ds(c * N + s * WR + t, sz)], sems[idx % 2]).wait()

    @pl.when(s == NS - 1)
    def _():
        tail = N - NS * WR
        pltpu.sync_copy(acc.at[pl.ds(NS * WR, tail)],
                        rows0.at[pl.ds(0, tail)])
        pltpu.sync_copy(rows0.at[pl.ds(0, tail)],
                        out_hbm.at[pl.ds(c * N + NS * WR, tail)])

    plsc.subcore_barrier()


def _segsum2_body(p0_hbm, p1_hbm, srcoff0_hbm, srcoff1_hbm, dst0_hbm,
                  dst1_hbm, zeros_hbm, out0_hbm, out1_hbm,
                  src_v, dst_v, rows0, rows1, acc, semz, sem0, sem1,
                  sems0, sems1, semh0, semh1):
    c = lax.axis_index("c")
    s = lax.axis_index("s")
    args = (src_v, dst_v, (rows0, rows1), acc, semz, (sem0, sem1),
            (sems0, sems1), (semh0, semh1), c, s)
    _one_relation(p0_hbm, srcoff0_hbm, dst0_hbm, zeros_hbm, out0_hbm, *args)
    _one_relation(p1_hbm, srcoff1_hbm, dst1_hbm, zeros_hbm, out1_hbm, *args)


@functools.lru_cache(maxsize=None)
def _segsum2_kernel():
    return pl.kernel(
        _segsum2_body,
        out_type=[jax.ShapeDtypeStruct((2 * N, HH), jnp.float32),
                  jax.ShapeDtypeStruct((2 * N, HH), jnp.float32)],
        mesh=_sc_mesh(),
        scratch_types=[
            pltpu.VMEM((HKCH, CH), jnp.int32),
            pltpu.VMEM((KCH, CH), jnp.int32),
            pltpu.VMEM((CH, HH), jnp.float32),
            pltpu.VMEM((CH, HH), jnp.float32),
            pltpu.VMEM_SHARED((NPAD, HH), jnp.float32),
            pltpu.SemaphoreType.DMA,
            pltpu.SemaphoreType.DMA,
            pltpu.SemaphoreType.DMA,
            pltpu.SemaphoreType.DMA,
            pltpu.SemaphoreType.DMA,
            pltpu.SemaphoreType.DMA,
            pltpu.SemaphoreType.DMA,
        ],
    )


def _segsum2(*args):
    return _segsum2_kernel()(*args)


KC2 = 2 * KCH


def _counts_body(dstcat_hbm, zeros_hbm, ones_hbm, out_hbm,
                 dst_v, ones_v, buf_v, acc, sem):
    c = lax.axis_index("c")
    s = lax.axis_index("s")
    pltpu.sync_copy(zeros_hbm.at[pl.ds(0, CH)], buf_v)
    zr = NPAD // NS
    zchunks = [(i * CH, CH) for i in range(zr // CH)] + [
        ((zr // CH) * CH, zr % CH)]
    for t, sz in zchunks:
        pltpu.sync_copy(buf_v.at[pl.ds(0, sz)],
                        acc.at[pl.ds(s * zr + t, sz)])
    pltpu.sync_copy(dstcat_hbm.at[c, s], dst_v)
    pltpu.sync_copy(ones_hbm, ones_v)
    plsc.subcore_barrier()

    def body(k, carry):
        pltpu.sync_copy(ones_v, acc.at[dst_v.at[k]], add=True)
        return carry

    lax.fori_loop(0, KCH, body, 0)
    plsc.subcore_barrier()
    chunks = [(i * CH, CH) for i in range(WR // CH)] + [
        ((WR // CH) * CH, WR % CH)]
    for t, sz in chunks:
        pltpu.sync_copy(acc.at[pl.ds(s * WR + t, sz)],
                        buf_v.at[pl.ds(0, sz)])
        pltpu.sync_copy(buf_v.at[pl.ds(0, sz)],
                        out_hbm.at[pl.ds(c * N + s * WR + t, sz)])

    @pl.when(s == NS - 1)
    def _():
        tail = N - NS * WR
        pltpu.sync_copy(acc.at[pl.ds(NS * WR, tail)],
                        buf_v.at[pl.ds(0, tail)])
        pltpu.sync_copy(buf_v.at[pl.ds(0, tail)],
                        out_hbm.at[pl.ds(c * N + NS * WR, tail)])


@functools.lru_cache(maxsize=None)
def _counts_kernel():
    return pl.kernel(
        _counts_body,
        out_type=jax.ShapeDtypeStruct((2 * N, HH), jnp.float32),
        mesh=_sc_mesh(),
        scratch_types=[
            pltpu.VMEM((KCH, CH), jnp.int32),
            pltpu.VMEM((CH, HH), jnp.float32),
            pltpu.VMEM((CH, HH), jnp.float32),
            pltpu.VMEM_SHARED((NPAD, HH), jnp.float32),
            pltpu.SemaphoreType.DMA,
        ],
    )


def _counts(*args):
    return _counts_kernel()(*args)


def _inproj_body(xu, xi, wu, bu, wi, bi, hu, hi):
    hu[...] = jnp.dot(xu[...], wu[...],
                      preferred_element_type=jnp.float32) + bu[...]
    hi[...] = jnp.dot(xi[...], wi[...],
                      preferred_element_type=jnp.float32) + bi[...]


def _input_proj(x_user, x_item, wu, bu, wi, bi):
    return pl.pallas_call(
        _inproj_body,
        grid=(NB,),
        in_specs=[
            pl.BlockSpec((RB, D_IN), lambda i: (i, 0)),
            pl.BlockSpec((RB, D_IN), lambda i: (i, 0)),
            pl.BlockSpec((D_IN, H), lambda i: (0, 0)),
            pl.BlockSpec((1, H), lambda i: (0, 0)),
            pl.BlockSpec((D_IN, H), lambda i: (0, 0)),
            pl.BlockSpec((1, H), lambda i: (0, 0)),
        ],
        out_specs=[
            pl.BlockSpec((RB, H), lambda i: (i, 0)),
            pl.BlockSpec((RB, H), lambda i: (i, 0)),
        ],
        out_shape=[
            jax.ShapeDtypeStruct((N, H), jnp.float32),
            jax.ShapeDtypeStruct((N, H), jnp.float32),
        ],
    )(x_user, x_item, wu, bu.reshape(1, H), wi, bi.reshape(1, H))


def _prep_body(hu, hi, wlu, wli, pu, pi):
    pu[...] = jnp.dot(hu[...], wlu[...], preferred_element_type=jnp.float32)
    pi[...] = jnp.dot(hi[...], wli[...], preferred_element_type=jnp.float32)


def _pre_p(hu, hi, wlu, wli):
    return pl.pallas_call(
        _prep_body,
        grid=(NB, 2),
        in_specs=[
            pl.BlockSpec((RB, H), lambda i, c: (i, 0)),
            pl.BlockSpec((RB, H), lambda i, c: (i, 0)),
            pl.BlockSpec((H, HH), lambda i, c: (0, c)),
            pl.BlockSpec((H, HH), lambda i, c: (0, c)),
        ],
        out_specs=[
            pl.BlockSpec((RB, HH), lambda i, c: (c * NB + i, 0)),
            pl.BlockSpec((RB, HH), lambda i, c: (c * NB + i, 0)),
        ],
        out_shape=[
            jax.ShapeDtypeStruct((2 * N, HH), jnp.float32),
            jax.ShapeDtypeStruct((2 * N, HH), jnp.float32),
        ],
    )(hu, hi, wlu, wli)


def _prer_body(hu, hi, wru, blu, wri, bli, ri, ru):
    ri[...] = jnp.dot(hi[...], wru[...],
                      preferred_element_type=jnp.float32) + blu[...]
    ru[...] = jnp.dot(hu[...], wri[...],
                      preferred_element_type=jnp.float32) + bli[...]


def _pre_r(hu, hi, wru, blu, wri, bli):
    return pl.pallas_call(
        _prer_body,
        grid=(NB,),
        in_specs=[
            pl.BlockSpec((RB, H), lambda i: (i, 0)),
            pl.BlockSpec((RB, H), lambda i: (i, 0)),
            pl.BlockSpec((H, H), lambda i: (0, 0)),
            pl.BlockSpec((1, H), lambda i: (0, 0)),
            pl.BlockSpec((H, H), lambda i: (0, 0)),
            pl.BlockSpec((1, H), lambda i: (0, 0)),
        ],
        out_specs=[
            pl.BlockSpec((RB, H), lambda i: (i, 0)),
            pl.BlockSpec((RB, H), lambda i: (i, 0)),
        ],
        out_shape=[
            jax.ShapeDtypeStruct((N, H), jnp.float32),
            jax.ShapeDtypeStruct((N, H), jnp.float32),
        ],
    )(hu, hi, wru, blu.reshape(1, H), wri, bli.reshape(1, H))


def _mean_add(s0, s1, cnt, r):
    rinv = 1.0 / jnp.maximum(cnt[...][:, :1], 1.0)
    return jnp.concatenate([s0[...], s1[...]], axis=1) * rinv + r[...]


def _posta_body(s0, s1, cnt, r, x, ssum, ssq):
    i = pl.program_id(0)
    xb = _mean_add(s0, s1, cnt, r)
    x[...] = xb

    @pl.when(i == 0)
    def _():
        ssum[...] = jnp.zeros_like(ssum)
        ssq[...] = jnp.zeros_like(ssq)

    ssum[...] += jnp.broadcast_to(jnp.sum(xb, 0, keepdims=True), (8, H))
    ssq[...] += jnp.broadcast_to(jnp.sum(xb * xb, 0, keepdims=True), (8, H))


def _post_a(s, cnt, r):
    stat = pl.BlockSpec((8, H), lambda i: (0, 0))
    return pl.pallas_call(
        _posta_body,
        grid=(NB,),
        in_specs=[
            pl.BlockSpec((RB, HH), lambda i: (i, 0)),
            pl.BlockSpec((RB, HH), lambda i: (NB + i, 0)),
            pl.BlockSpec((RB, HH), lambda i: (i, 0)),
            pl.BlockSpec((RB, H), lambda i: (i, 0)),
        ],
        out_specs=[
            pl.BlockSpec((RB, H), lambda i: (i, 0)),
            stat, stat,
        ],
        out_shape=[
            jax.ShapeDtypeStruct((N, H), jnp.float32),
            jax.ShapeDtypeStruct((8, H), jnp.float32),
            jax.ShapeDtypeStruct((8, H), jnp.float32),
        ],
    )(s, s, cnt, r)


def _bn_relu(x, ssum, ssq, g, b):
    m = ssum[...][:1, :] * (1.0 / N)
    v = ssq[...][:1, :] * (1.0 / N) - m * m
    y = (x[...] - m) * lax.rsqrt(v + 1e-5) * g[...] + b[...]
    return jnp.maximum(y, 0.0)


def _postb_body(x, ssum, ssq, g, b, h):
    h[...] = _bn_relu(x, ssum, ssq, g, b)


def _post_b(x, ssum, ssq, g, b):
    return pl.pallas_call(
        _postb_body,
        grid=(NB,),
        in_specs=[
            pl.BlockSpec((RB, H), lambda i: (i, 0)),
            pl.BlockSpec((8, H), lambda i: (0, 0)),
            pl.BlockSpec((8, H), lambda i: (0, 0)),
            pl.BlockSpec((1, H), lambda i: (0, 0)),
            pl.BlockSpec((1, H), lambda i: (0, 0)),
        ],
        out_specs=pl.BlockSpec((RB, H), lambda i: (i, 0)),
        out_shape=jax.ShapeDtypeStruct((N, H), jnp.float32),
    )(x, ssum, ssq, g.reshape(1, H), b.reshape(1, H))


def _final_body(s0, s1, cnt, r, w, b, o):
    x = _mean_add(s0, s1, cnt, r)
    o[...] = jnp.dot(x, w[...], preferred_element_type=jnp.float32) + b[...]


def _final(s, cnt, r, w, b):
    return pl.pallas_call(
        _final_body,
        grid=(NB,),
        in_specs=[
            pl.BlockSpec((RB, HH), lambda i: (i, 0)),
            pl.BlockSpec((RB, HH), lambda i: (NB + i, 0)),
            pl.BlockSpec((RB, HH), lambda i: (i, 0)),
            pl.BlockSpec((RB, H), lambda i: (i, 0)),
            pl.BlockSpec((H, H), lambda i: (0, 0)),
            pl.BlockSpec((1, H), lambda i: (0, 0)),
        ],
        out_specs=pl.BlockSpec((RB, H), lambda i: (i, 0)),
        out_shape=jax.ShapeDtypeStruct((N, H), jnp.float32),
    )(s, s, cnt, r, w, b.reshape(1, H))


def _prep_edges(ei):
    src = ei[0].astype(jnp.int32)
    dst = ei[1].astype(jnp.int32)
    pad = E_PAD - E
    src_p = jnp.concatenate([src, jnp.zeros((pad,), jnp.int32)])
    dst_p = jnp.concatenate([dst, jnp.full((pad,), N, jnp.int32)])
    srcoff = jnp.stack([src_p, src_p + N]).reshape(2, NS, KCH, CH)
    dst3 = dst_p.reshape(NS, KCH, CH)
    return srcoff, dst3, dst_p


def kernel(x_user, x_item, params, ei_u2i, ei_i2u):
    p = params
    srcoff_u2i, dst3_u2i, dstp_u2i = _prep_edges(ei_u2i)
    srcoff_i2u, dst3_i2u, dstp_i2u = _prep_edges(ei_i2u)
    dstcat = jnp.stack([dst3_u2i, dst3_i2u])

    zeros_seg = jnp.zeros((NPAD, HH), jnp.float32)
    ones_r = jnp.ones((CH, HH), jnp.float32)

    cnts = _counts(dstcat, zeros_seg, ones_r)
    cnt_i = cnts[:N]
    cnt_u = cnts[N:]

    h_u, h_i = _input_proj(x_user, x_item,
                           p['W_in_u'], p['b_in_u'], p['W_in_i'], p['b_in_i'])

    for l in range(3):
        pu, pi = _pre_p(h_u, h_i, p['Wl_u2i_%d' % l], p['Wl_i2u_%d' % l])
        s_i, s_u = _segsum2(pu, pi, srcoff_u2i, srcoff_i2u,
                            dst3_u2i, dst3_i2u, zeros_seg)
        r_i, r_u = _pre_r(
            h_u, h_i, p['Wr_u2i_%d' % l], p['bl_u2i_%d' % l],
            p['Wr_i2u_%d' % l], p['bl_i2u_%d' % l])
        if l < 2:
            x_i, sumi, sqi = _post_a(s_i, cnt_i, r_i)
            h_i = _post_b(x_i, sumi, sqi,
                          p['bn_g_i_%d' % l], p['bn_b_i_%d' % l])
            x_u, sumu, squ = _post_a(s_u, cnt_u, r_u)
            h_u = _post_b(x_u, sumu, squ,
                          p['bn_g_u_%d' % l], p['bn_b_u_%d' % l])
        else:
            out_i = _final(s_i, cnt_i, r_i, p['W_out_i'], p['b_out_i'])
            out_u = _final(s_u, cnt_u, r_u, p['W_out_u'], p['b_out_u'])
    return out_u, out_i


# speedup vs baseline: 1.0588x; 1.0588x over previous
import functools

import jax
import jax.numpy as jnp
from jax import lax
from jax.experimental import pallas as pl
from jax.experimental.pallas import tpu as pltpu
from jax.experimental.pallas import tpu_sc as plsc

N = 10000
E = 160000
D_IN = 128
H = 256
HH = 128
NC, NS = 2, 16
CH = 128
KCH = 80
HKCH = KCH // 2
E_PAD = NS * KCH * CH
NPAD = 10112
WR = 624
RB = 400
NB = N // RB

@functools.lru_cache(maxsize=None)
def _sc_mesh():
    return plsc.VectorSubcoreMesh(core_axis_name="c", subcore_axis_name="s",
                                  num_cores=NC, num_subcores=NS)


def _one_relation(p_hbm, srcoff_hbm, dst_hbm, zeros_hbm, out_hbm,
                  src_v, dst_v, rows, acc, semz, sems, ssems, hsems, c, s):
    rows0, rows1 = rows
    HC = CH // 2
    pltpu.sync_copy(zeros_hbm.at[pl.ds(0, CH)], rows0)
    zr = NPAD // NS
    zchunks = [(i * CH, CH) for i in range(zr // CH)] + [
        ((zr // CH) * CH, zr % CH)]
    for t, sz in zchunks:
        pltpu.async_copy(rows0.at[pl.ds(0, sz)],
                         acc.at[pl.ds(s * zr + t, sz)], semz)
    pltpu.sync_copy(dst_hbm.at[s], dst_v)
    for t, sz in zchunks:
        pltpu.make_async_copy(rows0.at[pl.ds(0, sz)],
                              acc.at[pl.ds(s * zr + t, sz)], semz).wait()
    plsc.subcore_barrier()

    def _gather(k, b):
        for q in range(2):
            pltpu.async_copy(p_hbm.at[src_v.at[k, pl.ds(q * HC, HC)]],
                             rows[b].at[pl.ds(q * HC, HC)],
                             (sems[b], hsems[b])[q])

    def _gwait(k, b):
        for q in range(2):
            pltpu.make_async_copy(p_hbm.at[src_v.at[k, pl.ds(q * HC, HC)]],
                                  rows[b].at[pl.ds(q * HC, HC)],
                                  (sems[b], hsems[b])[q]).wait()

    for h in range(2):
        pltpu.sync_copy(srcoff_hbm.at[c, s, pl.ds(h * HKCH, HKCH)], src_v)
        _gather(0, 0)

        def body(kk, carry):
            for b in range(2):
                k = 2 * kk + b
                _gwait(k, b)
                pltpu.async_copy(rows[b], acc.at[dst_v.at[h * HKCH + k]],
                                 ssems[b], add=True)

                @pl.when(k + 1 < HKCH)
                def _():
                    @pl.when(k >= 1)
                    def _():
                        pltpu.make_async_copy(
                            rows[1 - b],
                            acc.at[dst_v.at[h * HKCH + k - 1]],
                            ssems[1 - b]).wait()

                    _gather(k + 1, 1 - b)
            return carry

        lax.fori_loop(0, HKCH // 2, body, 0)
        for b, k in ((0, HKCH - 2), (1, HKCH - 1)):
            pltpu.make_async_copy(rows[b], acc.at[dst_v.at[h * HKCH + k]],
                                  ssems[b]).wait()
    plsc.subcore_barrier()
    chunks = [(i * CH, CH) for i in range(WR // CH)] + [
        ((WR // CH) * CH, WR % CH)]
    for idx, (t, sz) in enumerate(chunks):
        b = idx % 2
        if idx >= 2:
            pt, psz = chunks[idx - 2]
            pltpu.make_async_copy(
                rows[b].at[pl.ds(0, psz)],
                out_hbm.at[pl.ds(c * N + s * WR + pt, psz)], sems[b]).wait()
        pltpu.sync_copy(acc.at[pl.ds(s * WR + t, sz)],
                        rows[b].at[pl.ds(0, sz)])
        pltpu.async_copy(rows[b].at[pl.ds(0, sz)],
                         out_hbm.at[pl.ds(c * N + s * WR + t, sz)], sems[b])
    for idx in (len(chunks) - 2, len(chunks) - 1):
        t, sz = chunks[idx]
        pltpu.make_async_copy(
            rows[idx % 2].at[pl.ds(0, sz)],
            out_hbm.at[pl.ds(c * N + s * WR + t, sz)], sems[idx % 2]).wait()

    @pl.when(s == NS - 1)
    def _():
        tail = N - NS * WR
        pltpu.sync_copy(acc.at[pl.ds(NS * WR, tail)],
                        rows0.at[pl.ds(0, tail)])
        pltpu.sync_copy(rows0.at[pl.ds(0, tail)],
                        out_hbm.at[pl.ds(c * N + NS * WR, tail)])

    plsc.subcore_barrier()


def _segsum_body(p_hbm, srcoff_hbm, dst_hbm, zeros_hbm, out_hbm,
                 src_v, dst_v, rows0, rows1, acc, semz, sem0, sem1,
                 sems0, sems1, semh0, semh1):
    c = lax.axis_index("c")
    s = lax.axis_index("s")
    _one_relation(p_hbm, srcoff_hbm, dst_hbm, zeros_hbm, out_hbm,
                  src_v, dst_v, (rows0, rows1), acc, semz, (sem0, sem1),
                  (sems0, sems1), (semh0, semh1), c, s)


@functools.lru_cache(maxsize=None)
def _segsum_kernel():
    return pl.kernel(
        _segsum_body,
        out_type=jax.ShapeDtypeStruct((2 * N, HH), jnp.float32),
        mesh=_sc_mesh(),
        scratch_types=[
            pltpu.VMEM((HKCH, CH), jnp.int32),
            pltpu.VMEM((KCH, CH), jnp.int32),
            pltpu.VMEM((CH, HH), jnp.float32),
            pltpu.VMEM((CH, HH), jnp.float32),
            pltpu.VMEM_SHARED((NPAD, HH), jnp.float32),
            pltpu.SemaphoreType.DMA,
            pltpu.SemaphoreType.DMA,
            pltpu.SemaphoreType.DMA,
            pltpu.SemaphoreType.DMA,
            pltpu.SemaphoreType.DMA,
            pltpu.SemaphoreType.DMA,
            pltpu.SemaphoreType.DMA,
        ],
    )


def _segsum(*args):
    return _segsum_kernel()(*args)


KC2 = 2 * KCH


def _counts_body(dstcat_hbm, zeros_hbm, ones_hbm, out_hbm,
                 dst_v, ones_v, buf_v, acc, sem):
    c = lax.axis_index("c")
    s = lax.axis_index("s")
    pltpu.sync_copy(zeros_hbm.at[pl.ds(0, CH)], buf_v)
    zr = NPAD // NS
    zchunks = [(i * CH, CH) for i in range(zr // CH)] + [
        ((zr // CH) * CH, zr % CH)]
    for t, sz in zchunks:
        pltpu.sync_copy(buf_v.at[pl.ds(0, sz)],
                        acc.at[pl.ds(s * zr + t, sz)])
    pltpu.sync_copy(dstcat_hbm.at[c, s], dst_v)
    pltpu.sync_copy(ones_hbm, ones_v)
    plsc.subcore_barrier()

    def body(k, carry):
        pltpu.sync_copy(ones_v, acc.at[dst_v.at[k]], add=True)
        return carry

    lax.fori_loop(0, KCH, body, 0)
    plsc.subcore_barrier()
    chunks = [(i * CH, CH) for i in range(WR // CH)] + [
        ((WR // CH) * CH, WR % CH)]
    for t, sz in chunks:
        pltpu.sync_copy(acc.at[pl.ds(s * WR + t, sz)],
                        buf_v.at[pl.ds(0, sz)])
        pltpu.sync_copy(buf_v.at[pl.ds(0, sz)],
                        out_hbm.at[pl.ds(c * N + s * WR + t, sz)])

    @pl.when(s == NS - 1)
    def _():
        tail = N - NS * WR
        pltpu.sync_copy(acc.at[pl.ds(NS * WR, tail)],
                        buf_v.at[pl.ds(0, tail)])
        pltpu.sync_copy(buf_v.at[pl.ds(0, tail)],
                        out_hbm.at[pl.ds(c * N + NS * WR, tail)])


@functools.lru_cache(maxsize=None)
def _counts_kernel():
    return pl.kernel(
        _counts_body,
        out_type=jax.ShapeDtypeStruct((2 * N, HH), jnp.float32),
        mesh=_sc_mesh(),
        scratch_types=[
            pltpu.VMEM((KCH, CH), jnp.int32),
            pltpu.VMEM((CH, HH), jnp.float32),
            pltpu.VMEM((CH, HH), jnp.float32),
            pltpu.VMEM_SHARED((NPAD, HH), jnp.float32),
            pltpu.SemaphoreType.DMA,
        ],
    )


def _counts(*args):
    return _counts_kernel()(*args)


def _inproj_body(xu, xi, wu, bu, wi, bi, hu, hi):
    hu[...] = jnp.dot(xu[...], wu[...],
                      preferred_element_type=jnp.float32) + bu[...]
    hi[...] = jnp.dot(xi[...], wi[...],
                      preferred_element_type=jnp.float32) + bi[...]


def _input_proj(x_user, x_item, wu, bu, wi, bi):
    return pl.pallas_call(
        _inproj_body,
        grid=(NB,),
        in_specs=[
            pl.BlockSpec((RB, D_IN), lambda i: (i, 0)),
            pl.BlockSpec((RB, D_IN), lambda i: (i, 0)),
            pl.BlockSpec((D_IN, H), lambda i: (0, 0)),
            pl.BlockSpec((1, H), lambda i: (0, 0)),
            pl.BlockSpec((D_IN, H), lambda i: (0, 0)),
            pl.BlockSpec((1, H), lambda i: (0, 0)),
        ],
        out_specs=[
            pl.BlockSpec((RB, H), lambda i: (i, 0)),
            pl.BlockSpec((RB, H), lambda i: (i, 0)),
        ],
        out_shape=[
            jax.ShapeDtypeStruct((N, H), jnp.float32),
            jax.ShapeDtypeStruct((N, H), jnp.float32),
        ],
    )(x_user, x_item, wu, bu.reshape(1, H), wi, bi.reshape(1, H))


def _prep_body(hu, hi, wlu, wli, pu, pi):
    pu[...] = jnp.dot(hu[...], wlu[...], preferred_element_type=jnp.float32)
    pi[...] = jnp.dot(hi[...], wli[...], preferred_element_type=jnp.float32)


def _pre_p(hu, hi, wlu, wli):
    return pl.pallas_call(
        _prep_body,
        grid=(NB, 2),
        in_specs=[
            pl.BlockSpec((RB, H), lambda i, c: (i, 0)),
            pl.BlockSpec((RB, H), lambda i, c: (i, 0)),
            pl.BlockSpec((H, HH), lambda i, c: (0, c)),
            pl.BlockSpec((H, HH), lambda i, c: (0, c)),
        ],
        out_specs=[
            pl.BlockSpec((RB, HH), lambda i, c: (c * NB + i, 0)),
            pl.BlockSpec((RB, HH), lambda i, c: (c * NB + i, 0)),
        ],
        out_shape=[
            jax.ShapeDtypeStruct((2 * N, HH), jnp.float32),
            jax.ShapeDtypeStruct((2 * N, HH), jnp.float32),
        ],
    )(hu, hi, wlu, wli)


def _prer_body(hu, hi, wru, blu, wri, bli, ri, ru):
    ri[...] = jnp.dot(hi[...], wru[...],
                      preferred_element_type=jnp.float32) + blu[...]
    ru[...] = jnp.dot(hu[...], wri[...],
                      preferred_element_type=jnp.float32) + bli[...]


def _pre_r(hu, hi, wru, blu, wri, bli):
    return pl.pallas_call(
        _prer_body,
        grid=(NB,),
        in_specs=[
            pl.BlockSpec((RB, H), lambda i: (i, 0)),
            pl.BlockSpec((RB, H), lambda i: (i, 0)),
            pl.BlockSpec((H, H), lambda i: (0, 0)),
            pl.BlockSpec((1, H), lambda i: (0, 0)),
            pl.BlockSpec((H, H), lambda i: (0, 0)),
            pl.BlockSpec((1, H), lambda i: (0, 0)),
        ],
        out_specs=[
            pl.BlockSpec((RB, H), lambda i: (i, 0)),
            pl.BlockSpec((RB, H), lambda i: (i, 0)),
        ],
        out_shape=[
            jax.ShapeDtypeStruct((N, H), jnp.float32),
            jax.ShapeDtypeStruct((N, H), jnp.float32),
        ],
    )(hu, hi, wru, blu.reshape(1, H), wri, bli.reshape(1, H))


def _mean_add(s0, s1, cnt, r):
    rinv = 1.0 / jnp.maximum(cnt[...][:, :1], 1.0)
    return jnp.concatenate([s0[...], s1[...]], axis=1) * rinv + r[...]


def _posta_body(s0, s1, cnt, r, x, ssum, ssq):
    i = pl.program_id(0)
    xb = _mean_add(s0, s1, cnt, r)
    x[...] = xb

    @pl.when(i == 0)
    def _():
        ssum[...] = jnp.zeros_like(ssum)
        ssq[...] = jnp.zeros_like(ssq)

    ssum[...] += jnp.broadcast_to(jnp.sum(xb, 0, keepdims=True), (8, H))
    ssq[...] += jnp.broadcast_to(jnp.sum(xb * xb, 0, keepdims=True), (8, H))


def _post_a(s, cnt, r):
    stat = pl.BlockSpec((8, H), lambda i: (0, 0))
    return pl.pallas_call(
        _posta_body,
        grid=(NB,),
        in_specs=[
            pl.BlockSpec((RB, HH), lambda i: (i, 0)),
            pl.BlockSpec((RB, HH), lambda i: (NB + i, 0)),
            pl.BlockSpec((RB, HH), lambda i: (i, 0)),
            pl.BlockSpec((RB, H), lambda i: (i, 0)),
        ],
        out_specs=[
            pl.BlockSpec((RB, H), lambda i: (i, 0)),
            stat, stat,
        ],
        out_shape=[
            jax.ShapeDtypeStruct((N, H), jnp.float32),
            jax.ShapeDtypeStruct((8, H), jnp.float32),
            jax.ShapeDtypeStruct((8, H), jnp.float32),
        ],
    )(s, s, cnt, r)


def _bn_relu(x, ssum, ssq, g, b):
    m = ssum[...][:1, :] * (1.0 / N)
    v = ssq[...][:1, :] * (1.0 / N) - m * m
    y = (x[...] - m) * lax.rsqrt(v + 1e-5) * g[...] + b[...]
    return jnp.maximum(y, 0.0)


def _postb_body(x, ssum, ssq, g, b, h):
    h[...] = _bn_relu(x, ssum, ssq, g, b)


def _post_b(x, ssum, ssq, g, b):
    return pl.pallas_call(
        _postb_body,
        grid=(NB,),
        in_specs=[
            pl.BlockSpec((RB, H), lambda i: (i, 0)),
            pl.BlockSpec((8, H), lambda i: (0, 0)),
            pl.BlockSpec((8, H), lambda i: (0, 0)),
            pl.BlockSpec((1, H), lambda i: (0, 0)),
            pl.BlockSpec((1, H), lambda i: (0, 0)),
        ],
        out_specs=pl.BlockSpec((RB, H), lambda i: (i, 0)),
        out_shape=jax.ShapeDtypeStruct((N, H), jnp.float32),
    )(x, ssum, ssq, g.reshape(1, H), b.reshape(1, H))


def _final_body(s0, s1, cnt, r, w, b, o):
    x = _mean_add(s0, s1, cnt, r)
    o[...] = jnp.dot(x, w[...], preferred_element_type=jnp.float32) + b[...]


def _final(s, cnt, r, w, b):
    return pl.pallas_call(
        _final_body,
        grid=(NB,),
        in_specs=[
            pl.BlockSpec((RB, HH), lambda i: (i, 0)),
            pl.BlockSpec((RB, HH), lambda i: (NB + i, 0)),
            pl.BlockSpec((RB, HH), lambda i: (i, 0)),
            pl.BlockSpec((RB, H), lambda i: (i, 0)),
            pl.BlockSpec((H, H), lambda i: (0, 0)),
            pl.BlockSpec((1, H), lambda i: (0, 0)),
        ],
        out_specs=pl.BlockSpec((RB, H), lambda i: (i, 0)),
        out_shape=jax.ShapeDtypeStruct((N, H), jnp.float32),
    )(s, s, cnt, r, w, b.reshape(1, H))


def _prep_edges(ei):
    src = ei[0].astype(jnp.int32)
    dst = ei[1].astype(jnp.int32)
    pad = E_PAD - E
    src_p = jnp.concatenate([src, jnp.zeros((pad,), jnp.int32)])
    dst_p = jnp.concatenate([dst, jnp.full((pad,), N, jnp.int32)])
    srcoff = jnp.stack([src_p, src_p + N]).reshape(2, NS, KCH, CH)
    dst3 = dst_p.reshape(NS, KCH, CH)
    return srcoff, dst3, dst_p


def kernel(x_user, x_item, params, ei_u2i, ei_i2u):
    p = params
    srcoff_u2i, dst3_u2i, dstp_u2i = _prep_edges(ei_u2i)
    srcoff_i2u, dst3_i2u, dstp_i2u = _prep_edges(ei_i2u)
    dstcat = jnp.stack([dst3_u2i, dst3_i2u])

    zeros_seg = jnp.zeros((NPAD, HH), jnp.float32)
    ones_r = jnp.ones((CH, HH), jnp.float32)

    cnts = _counts(dstcat, zeros_seg, ones_r)
    cnt_i = cnts[:N]
    cnt_u = cnts[N:]

    h_u, h_i = _input_proj(x_user, x_item,
                           p['W_in_u'], p['b_in_u'], p['W_in_i'], p['b_in_i'])

    for l in range(3):
        pu, pi = _pre_p(h_u, h_i, p['Wl_u2i_%d' % l], p['Wl_i2u_%d' % l])
        s_i = _segsum(pu, srcoff_u2i, dst3_u2i, zeros_seg)
        s_u = _segsum(pi, srcoff_i2u, dst3_i2u, zeros_seg)
        r_i, r_u = _pre_r(
            h_u, h_i, p['Wr_u2i_%d' % l], p['bl_u2i_%d' % l],
            p['Wr_i2u_%d' % l], p['bl_i2u_%d' % l])
        if l < 2:
            x_i, sumi, sqi = _post_a(s_i, cnt_i, r_i)
            h_i = _post_b(x_i, sumi, sqi,
                          p['bn_g_i_%d' % l], p['bn_b_i_%d' % l])
            x_u, sumu, squ = _post_a(s_u, cnt_u, r_u)
            h_u = _post_b(x_u, sumu, squ,
                          p['bn_g_u_%d' % l], p['bn_b_u_%d' % l])
        else:
            out_i = _final(s_i, cnt_i, r_i, p['W_out_i'], p['b_out_i'])
            out_u = _final(s_u, cnt_u, r_u, p['W_out_u'], p['b_out_u'])
    return out_u, out_i


# speedup vs baseline: 1.1274x; 1.0648x over previous
import functools

import jax
import jax.numpy as jnp
from jax import lax
from jax.experimental import pallas as pl
from jax.experimental.pallas import tpu as pltpu
from jax.experimental.pallas import tpu_sc as plsc

N = 10000
E = 160000
D_IN = 128
H = 256
HH = 128
NC, NS = 2, 16
CH = 128
KCH = 80
HKCH = KCH // 2
E_PAD = NS * KCH * CH
NPAD = 10112
WR = 624
RB = 400
NB = N // RB

@functools.lru_cache(maxsize=None)
def _sc_mesh():
    return plsc.VectorSubcoreMesh(core_axis_name="c", subcore_axis_name="s",
                                  num_cores=NC, num_subcores=NS)


def _one_relation(p_hbm, srcoff_hbm, dst_hbm, zeros_hbm, out_hbm,
                  src_v, dst_v, rows, acc, semz, sems, ssems, hsems, c, s):
    rows0, rows1 = rows
    HC = CH // 2
    pltpu.sync_copy(zeros_hbm.at[pl.ds(0, CH)], rows0)
    zr = NPAD // NS
    zchunks = [(i * CH, CH) for i in range(zr // CH)] + [
        ((zr // CH) * CH, zr % CH)]
    for t, sz in zchunks:
        pltpu.async_copy(rows0.at[pl.ds(0, sz)],
                         acc.at[pl.ds(s * zr + t, sz)], semz)
    pltpu.sync_copy(dst_hbm.at[s], dst_v)
    for t, sz in zchunks:
        pltpu.make_async_copy(rows0.at[pl.ds(0, sz)],
                              acc.at[pl.ds(s * zr + t, sz)], semz).wait()
    plsc.subcore_barrier()

    def _gather(k, b):
        for q in range(2):
            pltpu.async_copy(p_hbm.at[src_v.at[k, pl.ds(q * HC, HC)]],
                             rows[b].at[pl.ds(q * HC, HC)],
                             (sems[b], hsems[b])[q])

    def _gwait(k, b):
        for q in range(2):
            pltpu.make_async_copy(p_hbm.at[src_v.at[k, pl.ds(q * HC, HC)]],
                                  rows[b].at[pl.ds(q * HC, HC)],
                                  (sems[b], hsems[b])[q]).wait()

    for h in range(2):
        pltpu.sync_copy(srcoff_hbm.at[c, s, pl.ds(h * HKCH, HKCH)], src_v)
        _gather(0, 0)

        def body(kk, carry):
            for b in range(2):
                k = 2 * kk + b
                _gwait(k, b)
                pltpu.async_copy(rows[b], acc.at[dst_v.at[h * HKCH + k]],
                                 ssems[b], add=True)

                @pl.when(k + 1 < HKCH)
                def _():
                    @pl.when(k >= 1)
                    def _():
                        pltpu.make_async_copy(
                            rows[1 - b],
                            acc.at[dst_v.at[h * HKCH + k - 1]],
                            ssems[1 - b]).wait()

                    _gather(k + 1, 1 - b)
            return carry

        lax.fori_loop(0, HKCH // 2, body, 0)
        for b, k in ((0, HKCH - 2), (1, HKCH - 1)):
            pltpu.make_async_copy(rows[b], acc.at[dst_v.at[h * HKCH + k]],
                                  ssems[b]).wait()
    plsc.subcore_barrier()
    chunks = [(i * CH, CH) for i in range(WR // CH)] + [
        ((WR // CH) * CH, WR % CH)]
    for idx, (t, sz) in enumerate(chunks):
        b = idx % 2
        if idx >= 2:
            pt, psz = chunks[idx - 2]
            pltpu.make_async_copy(
                rows[b].at[pl.ds(0, psz)],
                out_hbm.at[pl.ds(c * N + s * WR + pt, psz)], sems[b]).wait()
        pltpu.sync_copy(acc.at[pl.ds(s * WR + t, sz)],
                        rows[b].at[pl.ds(0, sz)])
        pltpu.async_copy(rows[b].at[pl.ds(0, sz)],
                         out_hbm.at[pl.ds(c * N + s * WR + t, sz)], sems[b])
    for idx in (len(chunks) - 2, len(chunks) - 1):
        t, sz = chunks[idx]
        pltpu.make_async_copy(
            rows[idx % 2].at[pl.ds(0, sz)],
            out_hbm.at[pl.ds(c * N + s * WR + t, sz)], sems[idx % 2]).wait()

    @pl.when(s == NS - 1)
    def _():
        tail = N - NS * WR
        pltpu.sync_copy(acc.at[pl.ds(NS * WR, tail)],
                        rows0.at[pl.ds(0, tail)])
        pltpu.sync_copy(rows0.at[pl.ds(0, tail)],
                        out_hbm.at[pl.ds(c * N + NS * WR, tail)])

    plsc.subcore_barrier()


def _segsum_body(p_hbm, srcoff_hbm, dst_hbm, zeros_hbm, out_hbm,
                 src_v, dst_v, rows0, rows1, acc, semz, sem0, sem1,
                 sems0, sems1, semh0, semh1):
    c = lax.axis_index("c")
    s = lax.axis_index("s")
    _one_relation(p_hbm, srcoff_hbm, dst_hbm, zeros_hbm, out_hbm,
                  src_v, dst_v, (rows0, rows1), acc, semz, (sem0, sem1),
                  (sems0, sems1), (semh0, semh1), c, s)


@functools.lru_cache(maxsize=None)
def _segsum_kernel():
    return pl.kernel(
        _segsum_body,
        out_type=jax.ShapeDtypeStruct((2 * N, HH), jnp.float32),
        mesh=_sc_mesh(),
        scratch_types=[
            pltpu.VMEM((HKCH, CH), jnp.int32),
            pltpu.VMEM((KCH, CH), jnp.int32),
            pltpu.VMEM((CH, HH), jnp.float32),
            pltpu.VMEM((CH, HH), jnp.float32),
            pltpu.VMEM_SHARED((NPAD, HH), jnp.float32),
            pltpu.SemaphoreType.DMA,
            pltpu.SemaphoreType.DMA,
            pltpu.SemaphoreType.DMA,
            pltpu.SemaphoreType.DMA,
            pltpu.SemaphoreType.DMA,
            pltpu.SemaphoreType.DMA,
            pltpu.SemaphoreType.DMA,
        ],
    )


def _segsum(*args):
    return _segsum_kernel()(*args)


KC2 = 2 * KCH


def _counts_body(dstcat_hbm, zeros_hbm, ones_hbm, out_hbm,
                 dst_v, ones_v, buf_v, acc, sem):
    c = lax.axis_index("c")
    s = lax.axis_index("s")
    pltpu.sync_copy(zeros_hbm.at[pl.ds(0, CH)], buf_v)
    zr = NPAD // NS
    zchunks = [(i * CH, CH) for i in range(zr // CH)] + [
        ((zr // CH) * CH, zr % CH)]
    for t, sz in zchunks:
        pltpu.sync_copy(buf_v.at[pl.ds(0, sz)],
                        acc.at[pl.ds(s * zr + t, sz)])
    pltpu.sync_copy(dstcat_hbm.at[c, s], dst_v)
    pltpu.sync_copy(ones_hbm, ones_v)
    plsc.subcore_barrier()

    def body(k, carry):
        pltpu.sync_copy(ones_v, acc.at[dst_v.at[k]], add=True)
        return carry

    lax.fori_loop(0, KCH, body, 0)
    plsc.subcore_barrier()
    chunks = [(i * CH, CH) for i in range(WR // CH)] + [
        ((WR // CH) * CH, WR % CH)]
    for t, sz in chunks:
        pltpu.sync_copy(acc.at[pl.ds(s * WR + t, sz)],
                        buf_v.at[pl.ds(0, sz)])
        pltpu.sync_copy(buf_v.at[pl.ds(0, sz)],
                        out_hbm.at[pl.ds(c * N + s * WR + t, sz)])

    @pl.when(s == NS - 1)
    def _():
        tail = N - NS * WR
        pltpu.sync_copy(acc.at[pl.ds(NS * WR, tail)],
                        buf_v.at[pl.ds(0, tail)])
        pltpu.sync_copy(buf_v.at[pl.ds(0, tail)],
                        out_hbm.at[pl.ds(c * N + NS * WR, tail)])


@functools.lru_cache(maxsize=None)
def _counts_kernel():
    return pl.kernel(
        _counts_body,
        out_type=jax.ShapeDtypeStruct((2 * N, HH), jnp.float32),
        mesh=_sc_mesh(),
        scratch_types=[
            pltpu.VMEM((KCH, CH), jnp.int32),
            pltpu.VMEM((CH, HH), jnp.float32),
            pltpu.VMEM((CH, HH), jnp.float32),
            pltpu.VMEM_SHARED((NPAD, HH), jnp.float32),
            pltpu.SemaphoreType.DMA,
        ],
    )


def _counts(*args):
    return _counts_kernel()(*args)


def _inproj_body(xu, xi, wu, bu, wi, bi, hu, hi):
    hu[...] = jnp.dot(xu[...], wu[...],
                      preferred_element_type=jnp.float32) + bu[...]
    hi[...] = jnp.dot(xi[...], wi[...],
                      preferred_element_type=jnp.float32) + bi[...]


def _input_proj(x_user, x_item, wu, bu, wi, bi):
    return pl.pallas_call(
        _inproj_body,
        grid=(NB,),
        in_specs=[
            pl.BlockSpec((RB, D_IN), lambda i: (i, 0)),
            pl.BlockSpec((RB, D_IN), lambda i: (i, 0)),
            pl.BlockSpec((D_IN, H), lambda i: (0, 0)),
            pl.BlockSpec((1, H), lambda i: (0, 0)),
            pl.BlockSpec((D_IN, H), lambda i: (0, 0)),
            pl.BlockSpec((1, H), lambda i: (0, 0)),
        ],
        out_specs=[
            pl.BlockSpec((RB, H), lambda i: (i, 0)),
            pl.BlockSpec((RB, H), lambda i: (i, 0)),
        ],
        out_shape=[
            jax.ShapeDtypeStruct((N, H), jnp.float32),
            jax.ShapeDtypeStruct((N, H), jnp.float32),
        ],
    )(x_user, x_item, wu, bu.reshape(1, H), wi, bi.reshape(1, H))


def _proj_body(h, wl, pp):
    pp[...] = jnp.dot(h[...], wl[...], preferred_element_type=jnp.float32)


def _proj(h, wl):
    return pl.pallas_call(
        _proj_body,
        grid=(NB, 2),
        in_specs=[
            pl.BlockSpec((RB, H), lambda i, c: (i, 0)),
            pl.BlockSpec((H, HH), lambda i, c: (0, c)),
        ],
        out_specs=pl.BlockSpec((RB, HH), lambda i, c: (c * NB + i, 0)),
        out_shape=jax.ShapeDtypeStruct((2 * N, HH), jnp.float32),
    )(h, wl)


def _prer_body(hu, hi, wru, blu, wri, bli, ri, ru):
    ri[...] = jnp.dot(hi[...], wru[...],
                      preferred_element_type=jnp.float32) + blu[...]
    ru[...] = jnp.dot(hu[...], wri[...],
                      preferred_element_type=jnp.float32) + bli[...]


def _pre_r(hu, hi, wru, blu, wri, bli):
    return pl.pallas_call(
        _prer_body,
        grid=(NB,),
        in_specs=[
            pl.BlockSpec((RB, H), lambda i: (i, 0)),
            pl.BlockSpec((RB, H), lambda i: (i, 0)),
            pl.BlockSpec((H, H), lambda i: (0, 0)),
            pl.BlockSpec((1, H), lambda i: (0, 0)),
            pl.BlockSpec((H, H), lambda i: (0, 0)),
            pl.BlockSpec((1, H), lambda i: (0, 0)),
        ],
        out_specs=[
            pl.BlockSpec((RB, H), lambda i: (i, 0)),
            pl.BlockSpec((RB, H), lambda i: (i, 0)),
        ],
        out_shape=[
            jax.ShapeDtypeStruct((N, H), jnp.float32),
            jax.ShapeDtypeStruct((N, H), jnp.float32),
        ],
    )(hu, hi, wru, blu.reshape(1, H), wri, bli.reshape(1, H))


def _mean_add(s0, s1, cnt, r):
    rinv = 1.0 / jnp.maximum(cnt[...][:, :1], 1.0)
    return jnp.concatenate([s0[...], s1[...]], axis=1) * rinv + r[...]


def _posta_body(s0, s1, cnt, r, x, ssum, ssq):
    i = pl.program_id(0)
    xb = _mean_add(s0, s1, cnt, r)
    x[...] = xb

    @pl.when(i == 0)
    def _():
        ssum[...] = jnp.zeros_like(ssum)
        ssq[...] = jnp.zeros_like(ssq)

    ssum[...] += jnp.broadcast_to(jnp.sum(xb, 0, keepdims=True), (8, H))
    ssq[...] += jnp.broadcast_to(jnp.sum(xb * xb, 0, keepdims=True), (8, H))


def _post_a(s, cnt, r):
    stat = pl.BlockSpec((8, H), lambda i: (0, 0))
    return pl.pallas_call(
        _posta_body,
        grid=(NB,),
        in_specs=[
            pl.BlockSpec((RB, HH), lambda i: (i, 0)),
            pl.BlockSpec((RB, HH), lambda i: (NB + i, 0)),
            pl.BlockSpec((RB, HH), lambda i: (i, 0)),
            pl.BlockSpec((RB, H), lambda i: (i, 0)),
        ],
        out_specs=[
            pl.BlockSpec((RB, H), lambda i: (i, 0)),
            stat, stat,
        ],
        out_shape=[
            jax.ShapeDtypeStruct((N, H), jnp.float32),
            jax.ShapeDtypeStruct((8, H), jnp.float32),
            jax.ShapeDtypeStruct((8, H), jnp.float32),
        ],
    )(s, s, cnt, r)


def _bn_relu(x, ssum, ssq, g, b):
    m = ssum[...][:1, :] * (1.0 / N)
    v = ssq[...][:1, :] * (1.0 / N) - m * m
    y = (x[...] - m) * lax.rsqrt(v + 1e-5) * g[...] + b[...]
    return jnp.maximum(y, 0.0)


def _postb_body(x, ssum, ssq, g, b, h):
    h[...] = _bn_relu(x, ssum, ssq, g, b)


def _post_b(x, ssum, ssq, g, b):
    return pl.pallas_call(
        _postb_body,
        grid=(NB,),
        in_specs=[
            pl.BlockSpec((RB, H), lambda i: (i, 0)),
            pl.BlockSpec((8, H), lambda i: (0, 0)),
            pl.BlockSpec((8, H), lambda i: (0, 0)),
            pl.BlockSpec((1, H), lambda i: (0, 0)),
            pl.BlockSpec((1, H), lambda i: (0, 0)),
        ],
        out_specs=pl.BlockSpec((RB, H), lambda i: (i, 0)),
        out_shape=jax.ShapeDtypeStruct((N, H), jnp.float32),
    )(x, ssum, ssq, g.reshape(1, H), b.reshape(1, H))


def _final_body(s0, s1, cnt, r, w, b, o):
    x = _mean_add(s0, s1, cnt, r)
    o[...] = jnp.dot(x, w[...], preferred_element_type=jnp.float32) + b[...]


def _final(s, cnt, r, w, b):
    return pl.pallas_call(
        _final_body,
        grid=(NB,),
        in_specs=[
            pl.BlockSpec((RB, HH), lambda i: (i, 0)),
            pl.BlockSpec((RB, HH), lambda i: (NB + i, 0)),
            pl.BlockSpec((RB, HH), lambda i: (i, 0)),
            pl.BlockSpec((RB, H), lambda i: (i, 0)),
            pl.BlockSpec((H, H), lambda i: (0, 0)),
            pl.BlockSpec((1, H), lambda i: (0, 0)),
        ],
        out_specs=pl.BlockSpec((RB, H), lambda i: (i, 0)),
        out_shape=jax.ShapeDtypeStruct((N, H), jnp.float32),
    )(s, s, cnt, r, w, b.reshape(1, H))


def _prep_edges(ei):
    src = ei[0].astype(jnp.int32)
    dst = ei[1].astype(jnp.int32)
    pad = E_PAD - E
    src_p = jnp.concatenate([src, jnp.zeros((pad,), jnp.int32)])
    dst_p = jnp.concatenate([dst, jnp.full((pad,), N, jnp.int32)])
    srcoff = jnp.stack([src_p, src_p + N]).reshape(2, NS, KCH, CH)
    dst3 = dst_p.reshape(NS, KCH, CH)
    return srcoff, dst3, dst_p


def kernel(x_user, x_item, params, ei_u2i, ei_i2u):
    p = params
    srcoff_u2i, dst3_u2i, dstp_u2i = _prep_edges(ei_u2i)
    srcoff_i2u, dst3_i2u, dstp_i2u = _prep_edges(ei_i2u)
    dstcat = jnp.stack([dst3_u2i, dst3_i2u])

    zeros_seg = jnp.zeros((NPAD, HH), jnp.float32)
    ones_r = jnp.ones((CH, HH), jnp.float32)

    cnts = _counts(dstcat, zeros_seg, ones_r)
    cnt_i = cnts[:N]
    cnt_u = cnts[N:]

    h_u, h_i = _input_proj(x_user, x_item,
                           p['W_in_u'], p['b_in_u'], p['W_in_i'], p['b_in_i'])

    for l in range(3):
        first_u2i = (l % 2 == 0)
        order = ('u2i', 'i2u') if first_u2i else ('i2u', 'u2i')
        seg = {}
        for rel in order:
            if rel == 'u2i':
                pp = _proj(h_u, p['Wl_u2i_%d' % l])
                seg['i'] = _segsum(pp, srcoff_u2i, dst3_u2i, zeros_seg)
            else:
                pp = _proj(h_i, p['Wl_i2u_%d' % l])
                seg['u'] = _segsum(pp, srcoff_i2u, dst3_i2u, zeros_seg)
        s_i, s_u = seg['i'], seg['u']
        r_i, r_u = _pre_r(
            h_u, h_i, p['Wr_u2i_%d' % l], p['bl_u2i_%d' % l],
            p['Wr_i2u_%d' % l], p['bl_i2u_%d' % l])
        if l < 2:
            x_i, sumi, sqi = _post_a(s_i, cnt_i, r_i)
            h_i = _post_b(x_i, sumi, sqi,
                          p['bn_g_i_%d' % l], p['bn_b_i_%d' % l])
            x_u, sumu, squ = _post_a(s_u, cnt_u, r_u)
            h_u = _post_b(x_u, sumu, squ,
                          p['bn_g_u_%d' % l], p['bn_b_u_%d' % l])
        else:
            out_i = _final(s_i, cnt_i, r_i, p['W_out_i'], p['b_out_i'])
            out_u = _final(s_u, cnt_u, r_u, p['W_out_u'], p['b_out_u'])
    return out_u, out_i
